# baseline (speedup 1.0000x reference)
import jax
import jax.numpy as jnp
from jax import lax
from jax.experimental import pallas as pl
from jax.experimental.pallas import tpu as pltpu
from jax.experimental.pallas import tpu_sc as plsc

_N_CLUSTERS = 50
_CLUSTER = 2000
_TOT = 3200
_T = 128
_D = 8
_K = 64
_CPB = 5
_NBLK = _N_CLUSTERS // _CPB
_NW = 32
_RPW = _TOT // _NW


def _sc_gather(x_hbm, idx_hbm, out_hbm, idx_v, rows_v, sem):
    wid = lax.axis_index("s") * 2 + lax.axis_index("c")
    pltpu.sync_copy(idx_hbm.at[wid], idx_v)
    pltpu.async_copy(x_hbm.at[idx_v], rows_v, sem).wait()
    pltpu.sync_copy(rows_v, out_hbm.at[wid])


def _gather_rows(x2d, gidx):
    mesh = plsc.VectorSubcoreMesh(core_axis_name="c", subcore_axis_name="s")
    f = pl.kernel(
        _sc_gather,
        mesh=mesh,
        out_type=jax.ShapeDtypeStruct((_NW, _RPW, _T), jnp.float32),
        scratch_types=[
            pltpu.VMEM((_RPW,), jnp.int32),
            pltpu.VMEM((_RPW, _T), jnp.float32),
            pltpu.SemaphoreType.DMA,
        ],
    )
    return f(x2d, gidx).reshape(_TOT, _T)


def _recur(t_ref, xg_ref, sgf_ref, dgf_ref, slf_ref, dlg_ref, sg2_ref,
           cis_ref, olog, m_all, msd_all, xsc):
    tm = t_ref[...]
    olog[...] = jnp.zeros_like(olog)
    m_all[...] = (dgf_ref[...] == sgf_ref[...]).astype(jnp.bfloat16)
    msd_all[...] = (slf_ref[...] == dlg_ref[...]).astype(jnp.bfloat16)
    xsc[...] = jnp.dot(xg_ref[...], tm, preferred_element_type=jnp.float32)

    def body(t, carry):
        b0 = t * 2 * _K
        b1 = b0 + _K
        o = olog[...]
        m0 = m_all[pl.ds(b0, _K), :]
        m1 = m_all[pl.ds(b1, _K), :]
        inc0 = jnp.dot(m0, o, preferred_element_type=jnp.float32)
        inc1p = jnp.dot(m1, o, preferred_element_type=jnp.float32)

        convinc0 = jnp.dot(inc0, tm, preferred_element_type=jnp.float32)
        out0 = xsc[pl.ds(b0, _K), :] + jnp.dot(
            msd_all[pl.ds(b0, _K), :], convinc0,
            preferred_element_type=jnp.float32)

        loc = (dgf_ref[pl.ds(b1, _K), :] == sg2_ref[0, pl.ds(2 * t, 1), :]
               ).astype(jnp.bfloat16)
        inc1 = inc1p + jnp.dot(loc, out0, preferred_element_type=jnp.float32)
        convinc1 = jnp.dot(inc1, tm, preferred_element_type=jnp.float32)
        out1 = xsc[pl.ds(b1, _K), :] + jnp.dot(
            msd_all[pl.ds(b1, _K), :], convinc1,
            preferred_element_type=jnp.float32)

        olog[pl.ds(b0, _K), :] = out0.astype(jnp.bfloat16)
        olog[pl.ds(b1, _K), :] = out1.astype(jnp.bfloat16)
        cis_ref[pl.ds(b0, _K), :] = convinc0
        cis_ref[pl.ds(b1, _K), :] = convinc1
        return carry

    lax.fori_loop(0, _N_CLUSTERS // 2, body, 0)


def _dense(x_ref, t_ref, dl_ref, cis_ref, y_ref):
    i = pl.program_id(0)
    tm = t_ref[...]
    for j in range(_CPB):
        dl_c = dl_ref[0, pl.ds(i * _CPB + j, 1), :]
        ohd = (lax.broadcasted_iota(jnp.int32, (_CLUSTER, _K), 0)
               == dl_c).astype(jnp.bfloat16)
        corr = jnp.dot(ohd, cis_ref[0, pl.ds(j * _K, _K), :],
                       preferred_element_type=jnp.float32)
        yj = jnp.dot(x_ref[0, pl.ds(j * _CLUSTER, _CLUSTER), :], tm,
                     preferred_element_type=jnp.float32)
        y_ref[0, pl.ds(j * _CLUSTER, _CLUSTER), :] = yj + corr


def _toeplitz(fir):
    idx = jnp.arange(_T)
    diff = idx[None, :] - idx[:, None]
    mask = (diff >= 0) & (diff < _D)
    return jnp.where(mask, fir[jnp.clip(diff, 0, _D - 1)], 0.0).astype(jnp.float32)


def kernel(x, kernel, dst_local, dst_gidx, src_local, src_gidx):
    fir = kernel
    tmat = _toeplitz(fir)

    sl32 = src_local.astype(jnp.int32)
    gidx = (jnp.arange(_N_CLUSTERS, dtype=jnp.int32)[:, None] * _CLUSTER
            + sl32).reshape(_NW, _RPW)
    xg = _gather_rows(x.reshape(_N_CLUSTERS * _CLUSTER, _T), gidx)

    sgf = src_gidx.astype(jnp.int32).reshape(1, _TOT)
    dgf = dst_gidx.astype(jnp.int32).reshape(_TOT, 1)
    slf = sl32.reshape(_TOT, 1)
    dlg = jnp.repeat(dst_local.astype(jnp.int32), _K, axis=0)
    sg2 = src_gidx.astype(jnp.int32).reshape(1, _N_CLUSTERS, _K)
    dl3 = dst_local.astype(jnp.int32).reshape(1, _N_CLUSTERS, _K)

    cis = pl.pallas_call(
        _recur,
        grid=(1,),
        in_specs=[
            pl.BlockSpec((_T, _T), lambda i: (0, 0)),
            pl.BlockSpec((_TOT, _T), lambda i: (0, 0)),
            pl.BlockSpec((1, _TOT), lambda i: (0, 0)),
            pl.BlockSpec((_TOT, 1), lambda i: (0, 0)),
            pl.BlockSpec((_TOT, 1), lambda i: (0, 0)),
            pl.BlockSpec((_TOT, _K), lambda i: (0, 0)),
            pl.BlockSpec((1, _N_CLUSTERS, _K), lambda i: (0, 0, 0)),
        ],
        out_specs=pl.BlockSpec((_TOT, _T), lambda i: (0, 0)),
        out_shape=jax.ShapeDtypeStruct((_TOT, _T), jnp.float32),
        scratch_shapes=[
            pltpu.VMEM((_TOT, _T), jnp.bfloat16),
            pltpu.VMEM((_TOT, _TOT), jnp.bfloat16),
            pltpu.VMEM((_TOT, _K), jnp.bfloat16),
            pltpu.VMEM((_TOT, _T), jnp.float32),
        ],
        compiler_params=pltpu.CompilerParams(
            dimension_semantics=("arbitrary",),
        ),
    )(tmat, xg, sgf, dgf, slf, dlg, sg2)

    cis3 = cis.reshape(_NBLK, _CPB * _K, _T)
    y = pl.pallas_call(
        _dense,
        grid=(_NBLK,),
        in_specs=[
            pl.BlockSpec((1, _CPB * _CLUSTER, _T), lambda i: (0, i, 0)),
            pl.BlockSpec((_T, _T), lambda i: (0, 0)),
            pl.BlockSpec((1, _N_CLUSTERS, _K), lambda i: (0, 0, 0)),
            pl.BlockSpec((1, _CPB * _K, _T), lambda i: (i, 0, 0)),
        ],
        out_specs=pl.BlockSpec((1, _CPB * _CLUSTER, _T), lambda i: (0, i, 0)),
        out_shape=jax.ShapeDtypeStruct(x.shape, jnp.float32),
        compiler_params=pltpu.CompilerParams(
            dimension_semantics=("arbitrary",),
        ),
    )(x, tmat, dl3, cis3)
    return y
